# baseline (speedup 1.0000x reference)
import functools

import jax
import jax.numpy as jnp
from jax import lax
from jax.experimental import pallas as pl
from jax.experimental.pallas import tpu as pltpu
from jax.experimental.pallas import tpu_sc as plsc

_K = 64
_W = 384
_RTC = 256
_BH = 16
_L = 16
_G = 6


def _bf16_rne(x):
    xi = lax.bitcast_convert_type(x, jnp.int32)
    r = (xi + 0x7FFF + ((xi >> 16) & 1)) & jnp.int32(-65536)
    return lax.bitcast_convert_type(r, jnp.float32)


def _tc_body(cn_ref, x_ref, o_ref):
    r0 = x_ref[0, 0]
    g0 = x_ref[0, 1]
    b0 = x_ref[0, 2]
    nrm = jnp.sqrt(r0 * r0 + g0 * g0 + b0 * b0)
    r = _bf16_rne(r0 / nrm)
    g = _bf16_rne(g0 / nrm)
    b = _bf16_rne(b0 / nrm)
    best_s = jnp.full(r.shape, -1.0, jnp.float32)
    best_r = jnp.zeros(r.shape, jnp.float32)
    best_g = jnp.zeros(r.shape, jnp.float32)
    best_b = jnp.zeros(r.shape, jnp.float32)
    for k in range(_K):
        s = r * cn_ref[k, 0] + g * cn_ref[k, 1] + b * cn_ref[k, 2]
        m = s > best_s
        best_s = jnp.where(m, s, best_s)
        best_r = jnp.where(m, cn_ref[k, 3], best_r)
        best_g = jnp.where(m, cn_ref[k, 4], best_g)
        best_b = jnp.where(m, cn_ref[k, 5], best_b)
    nz = (r0 + g0 + b0) > 0.0
    zero = jnp.zeros(r.shape, jnp.float32)
    o_ref[0, 0] = jnp.where(nz, best_r, zero)
    o_ref[0, 1] = jnp.where(nz, best_g, zero)
    o_ref[0, 2] = jnp.where(nz, best_b, zero)


def _nr_rsqrt(n2):
    i = lax.bitcast_convert_type(n2, jnp.int32)
    y = lax.bitcast_convert_type(jnp.int32(0x5F3759DF) - (i >> 1),
                                 jnp.float32)
    h = 0.5 * n2
    for _ in range(4):
        y = y * (1.5 - h * y * y)
    return y


def _sc_body(nrw, x_hbm, cns_hbm, tbl_hbm, out_hbm,
             rv, gv, bv, cs_v, tbl_v, ovr, ovg, ovb):
    wid = lax.axis_index("s") * 2 + lax.axis_index("c")
    b = wid // 8
    j = wid % 8
    row0 = _RTC + j * nrw

    pltpu.sync_copy(cns_hbm, cs_v)
    pltpu.sync_copy(tbl_hbm, tbl_v)
    pltpu.sync_copy(x_hbm.at[b, 0, pl.ds(row0, nrw), :], rv)
    pltpu.sync_copy(x_hbm.at[b, 1, pl.ds(row0, nrw), :], gv)
    pltpu.sync_copy(x_hbm.at[b, 2, pl.ds(row0, nrw), :], bv)

    def body(it, _):
        row = it >> 2
        lane0 = (it & 3) * (_G * _L)
        lanes = [lane0 + t * _L for t in range(_G)]
        r0 = [rv[row, pl.ds(l, _L)] for l in lanes]
        g0 = [gv[row, pl.ds(l, _L)] for l in lanes]
        b0 = [bv[row, pl.ds(l, _L)] for l in lanes]
        r, g, bb, bs, bi, nz = [], [], [], [], [], []
        for t in range(_G):
            inv = _nr_rsqrt(r0[t] * r0[t] + g0[t] * g0[t] + b0[t] * b0[t])
            r.append(_bf16_rne(r0[t] * inv))
            g.append(_bf16_rne(g0[t] * inv))
            bb.append(_bf16_rne(b0[t] * inv))
            nz.append((r0[t] + g0[t] + b0[t]) > 0.0)
            bs.append(jnp.full((_L,), -1.0, jnp.float32))
            bi.append(jnp.zeros((_L,), jnp.int32))
        for k in range(_K):
            cr = cs_v[pl.ds(48 * k, _L)]
            cg = cs_v[pl.ds(48 * k + 16, _L)]
            cb = cs_v[pl.ds(48 * k + 32, _L)]
            for t in range(_G):
                s = r[t] * cr + g[t] * cg + bb[t] * cb
                m = s > bs[t]
                bs[t] = jnp.where(m, s, bs[t])
                bi[t] = jnp.where(m, jnp.int32(k), bi[t])
        for t in range(_G):
            idx = jnp.where(nz[t], bi[t], jnp.int32(_K))
            for c, ov in ((0, ovr), (1, ovg), (2, ovb)):
                ov[row, pl.ds(lanes[t], _L)] = plsc.load_gather(
                    tbl_v, [idx + jnp.int32(72 * c)])
        return 0

    lax.fori_loop(0, nrw * 4, body, 0)
    orow0 = j * nrw
    pltpu.sync_copy(ovr, out_hbm.at[b, 0, pl.ds(orow0, nrw), :])
    pltpu.sync_copy(ovg, out_hbm.at[b, 1, pl.ds(orow0, nrw), :])
    pltpu.sync_copy(ovb, out_hbm.at[b, 2, pl.ds(orow0, nrw), :])


def kernel(rgb_mask, colors):
    B, C, H, W = rgb_mask.shape
    a_norm = jnp.linalg.norm(colors, ord=2, axis=-1)
    cn = colors / a_norm[:, None]
    cnr = _bf16_rne(cn)

    rrows = H - _RTC
    nrw = rrows // 8
    tbl = jnp.concatenate([cn, jnp.zeros((8, 3), jnp.float32)], axis=0)
    tbl = tbl.T.reshape(3 * 72)
    cns_splat = jnp.broadcast_to(cnr.reshape(192)[:, None], (192, _L))
    mesh = plsc.VectorSubcoreMesh(core_axis_name="c", subcore_axis_name="s")
    sc_out = pl.kernel(
        functools.partial(_sc_body, nrw),
        mesh=mesh,
        out_type=jax.ShapeDtypeStruct((B, C, rrows, W), jnp.float32),
        scratch_types=[
            pltpu.VMEM((nrw, _W), jnp.float32),
            pltpu.VMEM((nrw, _W), jnp.float32),
            pltpu.VMEM((nrw, _W), jnp.float32),
            pltpu.VMEM((192 * _L,), jnp.float32),
            pltpu.VMEM((3 * 72,), jnp.float32),
            pltpu.VMEM((nrw, _W), jnp.float32),
            pltpu.VMEM((nrw, _W), jnp.float32),
            pltpu.VMEM((nrw, _W), jnp.float32),
        ],
        compiler_params=pltpu.CompilerParams(needs_layout_passes=False),
    )(rgb_mask, cns_splat.reshape(192 * _L), tbl)

    tc_out = pl.pallas_call(
        _tc_body,
        grid=(B, _RTC // _BH),
        in_specs=[
            pl.BlockSpec(memory_space=pltpu.SMEM),
            pl.BlockSpec((1, C, _BH, W), lambda i, j: (i, 0, j, 0)),
        ],
        out_specs=pl.BlockSpec((1, C, _BH, W), lambda i, j: (i, 0, j, 0)),
        out_shape=jax.ShapeDtypeStruct((B, C, H, W), jnp.float32),
    )(jnp.concatenate([cnr, cn], axis=1), rgb_mask)

    return lax.dynamic_update_slice(tc_out, sc_out, (0, 0, _RTC, 0))


# speedup vs baseline: 1.0857x; 1.0857x over previous
import functools

import jax
import jax.numpy as jnp
from jax import lax
from jax.experimental import pallas as pl
from jax.experimental.pallas import tpu as pltpu
from jax.experimental.pallas import tpu_sc as plsc

_K = 64
_W = 384
_RTC = 256
_BH = 64
_L = 16
_G = 6


def _bf16_rne(x):
    xi = lax.bitcast_convert_type(x, jnp.int32)
    r = (xi + 0x7FFF + ((xi >> 16) & 1)) & jnp.int32(-65536)
    return lax.bitcast_convert_type(r, jnp.float32)


def _tc_body(cn_ref, x_ref, o_ref):
    r0 = x_ref[0, 0]
    g0 = x_ref[0, 1]
    b0 = x_ref[0, 2]
    nrm = jnp.sqrt(r0 * r0 + g0 * g0 + b0 * b0)
    r = _bf16_rne(r0 / nrm)
    g = _bf16_rne(g0 / nrm)
    b = _bf16_rne(b0 / nrm)
    best_s = jnp.full(r.shape, -1.0, jnp.float32)
    best_r = jnp.zeros(r.shape, jnp.float32)
    best_g = jnp.zeros(r.shape, jnp.float32)
    best_b = jnp.zeros(r.shape, jnp.float32)
    for k in range(_K):
        s = r * cn_ref[k, 0] + g * cn_ref[k, 1] + b * cn_ref[k, 2]
        m = s > best_s
        best_s = jnp.where(m, s, best_s)
        best_r = jnp.where(m, cn_ref[k, 3], best_r)
        best_g = jnp.where(m, cn_ref[k, 4], best_g)
        best_b = jnp.where(m, cn_ref[k, 5], best_b)
    nz = (r0 + g0 + b0) > 0.0
    zero = jnp.zeros(r.shape, jnp.float32)
    o_ref[0, 0] = jnp.where(nz, best_r, zero)
    o_ref[0, 1] = jnp.where(nz, best_g, zero)
    o_ref[0, 2] = jnp.where(nz, best_b, zero)


def _nr_rsqrt(n2):
    i = lax.bitcast_convert_type(n2, jnp.int32)
    y = lax.bitcast_convert_type(jnp.int32(0x5F3759DF) - (i >> 1),
                                 jnp.float32)
    h = 0.5 * n2
    for _ in range(4):
        y = y * (1.5 - h * y * y)
    return y


def _sc_body(nrw, x_hbm, cns_hbm, tbl_hbm, out_hbm,
             rv, gv, bv, cs_v, tbl_v, ovr, ovg, ovb):
    wid = lax.axis_index("s") * 2 + lax.axis_index("c")
    b = wid // 8
    j = wid % 8
    row0 = _RTC + j * nrw

    pltpu.sync_copy(cns_hbm, cs_v)
    pltpu.sync_copy(tbl_hbm, tbl_v)
    pltpu.sync_copy(x_hbm.at[b, 0, pl.ds(row0, nrw), :], rv)
    pltpu.sync_copy(x_hbm.at[b, 1, pl.ds(row0, nrw), :], gv)
    pltpu.sync_copy(x_hbm.at[b, 2, pl.ds(row0, nrw), :], bv)

    def body(it, _):
        row = it >> 2
        lane0 = (it & 3) * (_G * _L)
        lanes = [lane0 + t * _L for t in range(_G)]
        r0 = [rv[row, pl.ds(l, _L)] for l in lanes]
        g0 = [gv[row, pl.ds(l, _L)] for l in lanes]
        b0 = [bv[row, pl.ds(l, _L)] for l in lanes]
        r, g, bb, bs, bi, nz = [], [], [], [], [], []
        for t in range(_G):
            inv = _nr_rsqrt(r0[t] * r0[t] + g0[t] * g0[t] + b0[t] * b0[t])
            r.append(_bf16_rne(r0[t] * inv))
            g.append(_bf16_rne(g0[t] * inv))
            bb.append(_bf16_rne(b0[t] * inv))
            nz.append((r0[t] + g0[t] + b0[t]) > 0.0)
            bs.append(jnp.full((_L,), -1.0, jnp.float32))
            bi.append(jnp.zeros((_L,), jnp.int32))
        for k in range(_K):
            cr = cs_v[pl.ds(48 * k, _L)]
            cg = cs_v[pl.ds(48 * k + 16, _L)]
            cb = cs_v[pl.ds(48 * k + 32, _L)]
            for t in range(_G):
                s = r[t] * cr + g[t] * cg + bb[t] * cb
                m = s > bs[t]
                bs[t] = jnp.where(m, s, bs[t])
                bi[t] = jnp.where(m, jnp.int32(k), bi[t])
        for t in range(_G):
            idx = jnp.where(nz[t], bi[t], jnp.int32(_K))
            for c, ov in ((0, ovr), (1, ovg), (2, ovb)):
                ov[row, pl.ds(lanes[t], _L)] = plsc.load_gather(
                    tbl_v, [idx + jnp.int32(72 * c)])
        return 0

    lax.fori_loop(0, nrw * 4, body, 0)
    orow0 = j * nrw
    pltpu.sync_copy(ovr, out_hbm.at[b, 0, pl.ds(orow0, nrw), :])
    pltpu.sync_copy(ovg, out_hbm.at[b, 1, pl.ds(orow0, nrw), :])
    pltpu.sync_copy(ovb, out_hbm.at[b, 2, pl.ds(orow0, nrw), :])


def kernel(rgb_mask, colors):
    B, C, H, W = rgb_mask.shape
    a_norm = jnp.linalg.norm(colors, ord=2, axis=-1)
    cn = colors / a_norm[:, None]
    cnr = _bf16_rne(cn)

    rrows = H - _RTC
    nrw = rrows // 8
    tbl = jnp.concatenate([cn, jnp.zeros((8, 3), jnp.float32)], axis=0)
    tbl = tbl.T.reshape(3 * 72)
    cns_splat = jnp.broadcast_to(cnr.reshape(192)[:, None], (192, _L))
    mesh = plsc.VectorSubcoreMesh(core_axis_name="c", subcore_axis_name="s")
    sc_out = pl.kernel(
        functools.partial(_sc_body, nrw),
        mesh=mesh,
        out_type=jax.ShapeDtypeStruct((B, C, rrows, W), jnp.float32),
        scratch_types=[
            pltpu.VMEM((nrw, _W), jnp.float32),
            pltpu.VMEM((nrw, _W), jnp.float32),
            pltpu.VMEM((nrw, _W), jnp.float32),
            pltpu.VMEM((192 * _L,), jnp.float32),
            pltpu.VMEM((3 * 72,), jnp.float32),
            pltpu.VMEM((nrw, _W), jnp.float32),
            pltpu.VMEM((nrw, _W), jnp.float32),
            pltpu.VMEM((nrw, _W), jnp.float32),
        ],
        compiler_params=pltpu.CompilerParams(needs_layout_passes=False),
    )(rgb_mask, cns_splat.reshape(192 * _L), tbl)

    tc_out = pl.pallas_call(
        _tc_body,
        grid=(B, _RTC // _BH),
        in_specs=[
            pl.BlockSpec(memory_space=pltpu.SMEM),
            pl.BlockSpec((1, C, _BH, W), lambda i, j: (i, 0, j, 0)),
        ],
        out_specs=pl.BlockSpec((1, C, _BH, W), lambda i, j: (i, 0, j, 0)),
        out_shape=jax.ShapeDtypeStruct((B, C, H, W), jnp.float32),
    )(jnp.concatenate([cnr, cn], axis=1), rgb_mask)

    return lax.dynamic_update_slice(tc_out, sc_out, (0, 0, _RTC, 0))


# speedup vs baseline: 1.1037x; 1.0165x over previous
import functools

import jax
import jax.numpy as jnp
from jax import lax
from jax.experimental import pallas as pl
from jax.experimental.pallas import tpu as pltpu
from jax.experimental.pallas import tpu_sc as plsc

_K = 64
_W = 384
_RTC = 256
_BH = 64
_L = 16
_G = 6


def _bf16_rne(x):
    xi = lax.bitcast_convert_type(x, jnp.int32)
    r = (xi + 0x7FFF + ((xi >> 16) & 1)) & jnp.int32(-65536)
    return lax.bitcast_convert_type(r, jnp.float32)


def _tc_body(cn_ref, x_ref, o_ref):
    r0 = x_ref[0, 0]
    g0 = x_ref[0, 1]
    b0 = x_ref[0, 2]
    nrm = jnp.sqrt(r0 * r0 + g0 * g0 + b0 * b0)
    r = _bf16_rne(r0 / nrm)
    g = _bf16_rne(g0 / nrm)
    b = _bf16_rne(b0 / nrm)
    best_s = jnp.full(r.shape, -1.0, jnp.float32)
    best_r = jnp.zeros(r.shape, jnp.float32)
    best_g = jnp.zeros(r.shape, jnp.float32)
    best_b = jnp.zeros(r.shape, jnp.float32)
    for k in range(_K):
        s = r * cn_ref[k, 0] + g * cn_ref[k, 1] + b * cn_ref[k, 2]
        m = s > best_s
        best_s = jnp.where(m, s, best_s)
        best_r = jnp.where(m, cn_ref[k, 3], best_r)
        best_g = jnp.where(m, cn_ref[k, 4], best_g)
        best_b = jnp.where(m, cn_ref[k, 5], best_b)
    nz = (r0 + g0 + b0) > 0.0
    zero = jnp.zeros(r.shape, jnp.float32)
    o_ref[0, 0] = jnp.where(nz, best_r, zero)
    o_ref[0, 1] = jnp.where(nz, best_g, zero)
    o_ref[0, 2] = jnp.where(nz, best_b, zero)


def _nr_rsqrt(n2):
    i = lax.bitcast_convert_type(n2, jnp.int32)
    y = lax.bitcast_convert_type(jnp.int32(0x5F3759DF) - (i >> 1),
                                 jnp.float32)
    h = 0.5 * n2
    for _ in range(4):
        y = y * (1.5 - h * y * y)
    return y


def _sc_body(nrw, x_hbm, cns_hbm, tbl_hbm, out_hbm,
             rv, gv, bv, cs_v, tbl_v, ovr, ovg, ovb, dsem):
    wid = lax.axis_index("s") * 2 + lax.axis_index("c")
    b = wid // 8
    j = wid % 8
    row0 = _RTC + j * nrw

    cps = [
        pltpu.async_copy(x_hbm.at[b, 0, pl.ds(row0, nrw), :], rv, dsem),
        pltpu.async_copy(x_hbm.at[b, 1, pl.ds(row0, nrw), :], gv, dsem),
        pltpu.async_copy(x_hbm.at[b, 2, pl.ds(row0, nrw), :], bv, dsem),
        pltpu.async_copy(cns_hbm, cs_v, dsem),
        pltpu.async_copy(tbl_hbm, tbl_v, dsem),
    ]
    for cp in cps:
        cp.wait()

    def body(it, _):
        row = it >> 2
        lane0 = (it & 3) * (_G * _L)
        lanes = [lane0 + t * _L for t in range(_G)]
        r0 = [rv[row, pl.ds(l, _L)] for l in lanes]
        g0 = [gv[row, pl.ds(l, _L)] for l in lanes]
        b0 = [bv[row, pl.ds(l, _L)] for l in lanes]
        r, g, bb, bs, bi, nz = [], [], [], [], [], []
        for t in range(_G):
            inv = _nr_rsqrt(r0[t] * r0[t] + g0[t] * g0[t] + b0[t] * b0[t])
            r.append(_bf16_rne(r0[t] * inv))
            g.append(_bf16_rne(g0[t] * inv))
            bb.append(_bf16_rne(b0[t] * inv))
            nz.append((r0[t] + g0[t] + b0[t]) > 0.0)
            bs.append(jnp.full((_L,), -1.0, jnp.float32))
            bi.append(jnp.zeros((_L,), jnp.int32))
        for k in range(_K):
            cr = cs_v[pl.ds(48 * k, _L)]
            cg = cs_v[pl.ds(48 * k + 16, _L)]
            cb = cs_v[pl.ds(48 * k + 32, _L)]
            for t in range(_G):
                s = r[t] * cr + g[t] * cg + bb[t] * cb
                m = s > bs[t]
                bs[t] = jnp.where(m, s, bs[t])
                bi[t] = jnp.where(m, jnp.int32(k), bi[t])
        for t in range(_G):
            idx = jnp.where(nz[t], bi[t], jnp.int32(_K))
            for c, ov in ((0, ovr), (1, ovg), (2, ovb)):
                ov[row, pl.ds(lanes[t], _L)] = plsc.load_gather(
                    tbl_v, [idx + jnp.int32(72 * c)])
        return 0

    lax.fori_loop(0, nrw * 4, body, 0)
    orow0 = j * nrw
    pltpu.sync_copy(ovr, out_hbm.at[b, 0, pl.ds(orow0, nrw), :])
    pltpu.sync_copy(ovg, out_hbm.at[b, 1, pl.ds(orow0, nrw), :])
    pltpu.sync_copy(ovb, out_hbm.at[b, 2, pl.ds(orow0, nrw), :])


def kernel(rgb_mask, colors):
    B, C, H, W = rgb_mask.shape
    a_norm = jnp.linalg.norm(colors, ord=2, axis=-1)
    cn = colors / a_norm[:, None]
    cnr = _bf16_rne(cn)

    rrows = H - _RTC
    nrw = rrows // 8
    tbl = jnp.concatenate([cn, jnp.zeros((8, 3), jnp.float32)], axis=0)
    tbl = tbl.T.reshape(3 * 72)
    cns_splat = jnp.broadcast_to(cnr.reshape(192)[:, None], (192, _L))
    mesh = plsc.VectorSubcoreMesh(core_axis_name="c", subcore_axis_name="s")
    sc_out = pl.kernel(
        functools.partial(_sc_body, nrw),
        mesh=mesh,
        out_type=jax.ShapeDtypeStruct((B, C, rrows, W), jnp.float32),
        scratch_types=[
            pltpu.VMEM((nrw, _W), jnp.float32),
            pltpu.VMEM((nrw, _W), jnp.float32),
            pltpu.VMEM((nrw, _W), jnp.float32),
            pltpu.VMEM((192 * _L,), jnp.float32),
            pltpu.VMEM((3 * 72,), jnp.float32),
            pltpu.VMEM((nrw, _W), jnp.float32),
            pltpu.VMEM((nrw, _W), jnp.float32),
            pltpu.VMEM((nrw, _W), jnp.float32),
            pltpu.SemaphoreType.DMA,
        ],
        compiler_params=pltpu.CompilerParams(needs_layout_passes=False),
    )(rgb_mask, cns_splat.reshape(192 * _L), tbl)

    tc_out = pl.pallas_call(
        _tc_body,
        grid=(B, _RTC // _BH),
        in_specs=[
            pl.BlockSpec(memory_space=pltpu.SMEM),
            pl.BlockSpec((1, C, _BH, W), lambda i, j: (i, 0, j, 0)),
        ],
        out_specs=pl.BlockSpec((1, C, _BH, W), lambda i, j: (i, 0, j, 0)),
        out_shape=jax.ShapeDtypeStruct((B, C, H, W), jnp.float32),
    )(jnp.concatenate([cnr, cn], axis=1), rgb_mask)

    return lax.dynamic_update_slice(tc_out, sc_out, (0, 0, _RTC, 0))
